# Initial kernel scaffold; baseline (speedup 1.0000x reference)
import jax
import jax.numpy as jnp
from jax.experimental import pallas as pl


def kernel(x, edge_index, edge_weight, Wi, bi, Wh, bh, Wo, bo, Wm1, bm1, gamma, beta, Wm2, bm2):
    raise NotImplementedError("write your pallas kernel here")


# speedup vs baseline: 2.5047x; 2.5047x over previous
import functools

import numpy as np
import jax
import jax.numpy as jnp
from jax import lax
from jax.experimental import pallas as pl
from jax.experimental.pallas import tpu as pltpu
from jax.experimental.pallas import tpu_sc as plsc

N = 10000
E = 320000
N_PAD = 10240
NC, NS, LANES = 2, 16, 16
NW = NC * NS
CH = 128
E_PAD = 4096 * 79
TILE_E32 = E_PAD // NW
CHUNKS32 = TILE_E32 // CH
TILE_E16 = E_PAD // NS
CHUNKS16 = TILE_E16 // CH
NCHUNK = E_PAD // CH
DUMMY = N + 100

_MESH = dict(
    mesh=plsc.VectorSubcoreMesh(core_axis_name="c", subcore_axis_name="s"),
    compiler_params=pltpu.CompilerParams(needs_layout_passes=False),
)


NPT = N_PAD // NW
DEG_ROWS = 8
DEG_ITERS = NCHUNK // DEG_ROWS


def _deg_body(packed_hbm, deg_hbm, deg16, outbuf, pbuf):
    c = lax.axis_index("c")
    s = lax.axis_index("s")
    wid = s * NC + c
    lo = wid * NPT
    lane = lax.broadcasted_iota(jnp.int32, (16,), 0)

    def zero(i, carry):
        for l in range(16):
            deg16[l, pl.ds(i * 16, 16)] = jnp.zeros((16,), jnp.float32)
        return carry

    lax.fori_loop(0, NPT // 16, zero, 0)

    def chunk(k, carry):
        pltpu.sync_copy(packed_hbm.at[pl.ds(k * DEG_ROWS, DEG_ROWS)], pbuf)
        for q in range(DEG_ROWS):
            for j in range(CH // 16):
                r = pbuf[q, pl.ds(j * 16, 16)].astype(jnp.int32)
                cc = pbuf[q, pl.ds(CH + j * 16, 16)].astype(jnp.int32)
                ew = pbuf[q, pl.ds(2 * CH + j * 16, 16)]
                em = jnp.where(r != cc, ew, 0.0)
                ridx = r - lo
                m = (ridx >= 0) & (ridx < NPT)
                ridx = jnp.clip(ridx, 0, NPT - 1)
                plsc.addupdate_scatter(deg16, [lane, ridx], em, mask=m)
        return carry

    lax.fori_loop(0, DEG_ITERS, chunk, 0)

    def red(i, carry):
        slp = pl.ds(i * 16, 16)
        acc = deg16[0, slp]
        for l in range(1, 16):
            acc = acc + deg16[l, slp]
        outbuf[slp] = acc
        return carry

    lax.fori_loop(0, NPT // 16, red, 0)
    pltpu.sync_copy(outbuf, deg_hbm.at[pl.ds(lo, NPT)])


def _deg_call(packed):
    f = pl.kernel(
        _deg_body,
        out_type=jax.ShapeDtypeStruct((N_PAD,), jnp.float32),
        scratch_types=[
            pltpu.VMEM((16, NPT), jnp.float32),
            pltpu.VMEM((NPT,), jnp.float32),
            pltpu.VMEM((DEG_ROWS, 3 * CH), jnp.float32),
        ],
        **_MESH,
    )
    return f(packed)


def _dinv_body(p_ref, o_ref):
    deg = p_ref[...]
    o_ref[...] = jnp.where(deg > 0, 1.0 / jnp.sqrt(deg), 0.0)


def _dinv_call(partials):
    return pl.pallas_call(
        _dinv_body,
        out_shape=jax.ShapeDtypeStruct((1, N_PAD), jnp.float32),
    )(partials)


def _wnorm_body(packed_hbm, dinv_hbm, out_hbm, dbuf, pbuf):
    c = lax.axis_index("c")
    s = lax.axis_index("s")
    wid = s * NC + c
    pltpu.sync_copy(dinv_hbm, dbuf)
    cbase = wid * CHUNKS32

    def chunk(k, carry):
        pltpu.sync_copy(packed_hbm.at[cbase + k], pbuf)
        for j in range(CH // 16):
            r = pbuf[pl.ds(j * 16, 16)].astype(jnp.int32)
            cc = pbuf[pl.ds(CH + j * 16, 16)].astype(jnp.int32)
            ew = pbuf[pl.ds(2 * CH + j * 16, 16)]
            dr = plsc.load_gather(dbuf, [r])
            dc = plsc.load_gather(dbuf, [cc])
            wn = jnp.where(r != cc, -(dr * ew * dc), 0.0)
            pbuf[pl.ds(2 * CH + j * 16, 16)] = wn
        pltpu.sync_copy(pbuf, out_hbm.at[cbase + k])
        return carry

    lax.fori_loop(0, CHUNKS32, chunk, 0)


def _wnorm_call(packed, dinv):
    f = pl.kernel(
        _wnorm_body,
        out_type=jax.ShapeDtypeStruct((NCHUNK, 3 * CH), jnp.float32),
        scratch_types=[
            pltpu.VMEM((N_PAD,), jnp.float32),
            pltpu.VMEM((3 * CH,), jnp.float32),
        ],
        **_MESH,
    )
    return f(packed, dinv)


def _prop_body(dh, t_hbm, packed_hbm, out_hbm, acc, gbuf, pbuf, cidx, gidx, wbuf, zbuf, sem):
    c = lax.axis_index("c")
    s = lax.axis_index("s")
    dh16 = dh // 16
    rows_per_tile = N_PAD // NS
    rbase = s * rows_per_tile

    def zrow(i, carry):
        for j in range(dh16):
            zbuf[i, pl.ds(j * 16, 16)] = jnp.zeros((16,), jnp.float32)
        return carry

    lax.fori_loop(0, CH, zrow, 0)

    def zacc(i, carry):
        pltpu.sync_copy(zbuf, acc.at[pl.ds(rbase + i * CH, CH)])
        return carry

    lax.fori_loop(0, rows_per_tile // CH, zacc, 0)
    plsc.subcore_barrier()

    tbase = c * N_PAD
    cbase = s * CHUNKS16

    def chunk(k, carry):
        pltpu.sync_copy(packed_hbm.at[cbase + k], pbuf)
        for j in range(CH // 16):
            sl = pl.ds(j * 16, 16)
            gidx[sl] = pbuf[sl].astype(jnp.int32) + tbase
            cidx[sl] = pbuf[pl.ds(CH + j * 16, 16)].astype(jnp.int32)
            wbuf[sl] = pbuf[pl.ds(2 * CH + j * 16, 16)]
        pltpu.async_copy(t_hbm.at[gidx], gbuf, sem).wait()

        def edge(e, cr):
            eidx = jnp.broadcast_to(e, (16,)).astype(jnp.int32)
            we = plsc.load_gather(wbuf, [eidx])
            for j in range(dh16):
                sl = pl.ds(j * 16, 16)
                gbuf[e, sl] = gbuf[e, sl] * we
            return cr

        lax.fori_loop(0, CH, edge, 0)
        pltpu.sync_copy(gbuf, acc.at[cidx], add=True)
        return carry

    lax.fori_loop(0, CHUNKS16, chunk, 0)
    plsc.subcore_barrier()

    def wb(i, carry):
        r0 = rbase + i * CH
        pltpu.sync_copy(acc.at[pl.ds(r0, CH)], out_hbm.at[pl.ds(tbase + r0, CH)])
        return carry

    lax.fori_loop(0, rows_per_tile // CH, wb, 0)


def _prop_call(t_flat, packed, dh):
    f = pl.kernel(
        functools.partial(_prop_body, dh),
        out_type=jax.ShapeDtypeStruct((2 * N_PAD, dh), jnp.float32),
        scratch_types=[
            pltpu.VMEM_SHARED((N_PAD, dh), jnp.float32),
            pltpu.VMEM((CH, dh), jnp.float32),
            pltpu.VMEM((3 * CH,), jnp.float32),
            pltpu.VMEM((CH,), jnp.int32),
            pltpu.VMEM((CH,), jnp.int32),
            pltpu.VMEM((CH,), jnp.float32),
            pltpu.VMEM((CH, dh), jnp.float32),
            pltpu.SemaphoreType.DMA,
        ],
        **_MESH,
    )
    return f(t_flat, packed)


def _bd(x, w):
    return jnp.dot(x.astype(jnp.bfloat16), w.astype(jnp.bfloat16),
                   preferred_element_type=jnp.float32)


def _combine_body(din, dout, relu, h_ref, p1_ref, p2_ref, w_ref, b_ref, o_ref):
    dq = din // 2
    cat = lambda r: jnp.concatenate([r[0][:, :dq], r[1][:, :dq]], axis=1)
    h = cat(h_ref)
    p1 = cat(p1_ref)
    p2 = cat(p2_ref)
    out = _bd(h, w_ref[0])
    out = out + _bd(p1, w_ref[1])
    tx2 = 2.0 * p2 - h
    out = out + _bd(tx2, w_ref[2])
    out = out + b_ref[...]
    if relu:
        out = jnp.maximum(out, 0.0)
    o_ref[0] = out[:, : dout // 2]
    o_ref[1] = out[:, dout // 2:]


def _combine_call(h, p1, p2, W, b, din, dout, relu, br=640):
    grid = N_PAD // br
    io_spec = pl.BlockSpec((2, br, 128), lambda i: (0, i, 0))
    return pl.pallas_call(
        functools.partial(_combine_body, din, dout, relu),
        grid=(grid,),
        in_specs=[
            io_spec, io_spec, io_spec,
            pl.BlockSpec((3, din, dout), lambda i: (0, 0, 0)),
            pl.BlockSpec((1, dout), lambda i: (0, 0)),
        ],
        out_specs=pl.BlockSpec((2, br, dout // 2), lambda i: (0, i, 0)),
        out_shape=jax.ShapeDtypeStruct((2, N_PAD, dout // 2), jnp.float32),
    )(h, p1, p2, W, b.reshape(1, dout))


def _final_body(sqc, h_ref, p1_ref, p2_ref, w_ref, b_ref, wm1_ref, bm1_ref,
                gam_ref, bet_ref, wm2_ref, b2_ref, o_ref):
    cat = lambda r: jnp.concatenate([r[0], r[1]], axis=1)
    h = cat(h_ref)
    p1 = cat(p1_ref)
    p2 = cat(p2_ref)
    g = _bd(h, w_ref[0])
    g = g + _bd(p1, w_ref[1])
    tx2 = 2.0 * p2 - h
    g = g + _bd(tx2, w_ref[2])
    g = g + b_ref[...]
    m = _bd(g, wm1_ref[...]) + bm1_ref[...]
    m = m / sqc * gam_ref[...] + bet_ref[...]
    m = jnp.maximum(m, 0.0)
    z = _bd(m, wm2_ref[...]) + b2_ref[...]
    o_ref[...] = 1.0 / (1.0 + jnp.exp(-z))


def _final_call(h, p1, p2, W, b, wm1, bm1, gam, bet, wm2, b2, br=640):
    grid = N_PAD // br
    io_spec = pl.BlockSpec((2, br, 128), lambda i: (0, i, 0))
    full = lambda a, bdim: pl.BlockSpec((a, bdim), lambda i: (0, 0))
    sqc = float(np.sqrt(np.float32(1.0 + 1e-5)))
    return pl.pallas_call(
        functools.partial(_final_body, sqc),
        grid=(grid,),
        in_specs=[
            io_spec, io_spec, io_spec,
            pl.BlockSpec((3, 256, 128), lambda i: (0, 0, 0)),
            full(1, 128),
            full(128, 128), full(1, 128), full(1, 128), full(1, 128),
            full(128, 128), full(1, 128),
        ],
        out_specs=pl.BlockSpec((br, 128), lambda i: (i, 0)),
        out_shape=jax.ShapeDtypeStruct((N_PAD, 128), jnp.float32),
    )(h, p1, p2, W, b, wm1, bm1, gam, bet, wm2, b2)


def kernel(x, edge_index, edge_weight, Wi, bi, Wh, bh, Wo, bo, Wm1, bm1, gamma,
           beta, Wm2, bm2):
    f32 = jnp.float32
    row = edge_index[0]
    col = edge_index[1]
    pad = E_PAD - E
    row_p = jnp.pad(row, (0, pad))
    col_p = jnp.pad(col, (0, pad))
    ew_p = jnp.pad(edge_weight, (0, pad))

    def pack(r, c, w):
        return jnp.concatenate(
            [r.astype(f32).reshape(NCHUNK, CH),
             c.astype(f32).reshape(NCHUNK, CH),
             w.astype(f32).reshape(NCHUNK, CH)], axis=1)

    pr = jnp.argsort(row_p, stable=True)
    packed_rs = pack(row_p[pr], col_p[pr], ew_p[pr])
    pc = jnp.argsort(col_p, stable=True)
    packed_cs = pack(row_p[pc], col_p[pc], ew_p[pc])

    deg = _deg_call(packed_rs)
    dinv = _dinv_call(deg.reshape(1, N_PAD)).reshape(N_PAD)
    packed_n = _wnorm_call(packed_cs, dinv)

    def to_flat(a, dh):
        h0 = jnp.pad(a[:, :dh], ((0, N_PAD - N), (0, 128 - dh)))
        h1 = jnp.pad(a[:, dh:], ((0, N_PAD - N), (0, 128 - dh)))
        return jnp.concatenate([h0, h1], axis=0)

    h = to_flat(x, 64)
    p1 = _prop_call(h, packed_n, 128)
    p2 = _prop_call(p1, packed_n, 128)
    h = _combine_call(h.reshape(2, N_PAD, 128), p1.reshape(2, N_PAD, 128),
                      p2.reshape(2, N_PAD, 128), Wi, bi, 128, 256, True)
    h = h.reshape(2 * N_PAD, 128)
    for i in range(Wh.shape[0]):
        p1 = _prop_call(h, packed_n, 128)
        p2 = _prop_call(p1, packed_n, 128)
        h = _combine_call(h.reshape(2, N_PAD, 128), p1.reshape(2, N_PAD, 128),
                          p2.reshape(2, N_PAD, 128), Wh[i], bh[i], 256, 256, True)
        h = h.reshape(2 * N_PAD, 128)
    p1 = _prop_call(h, packed_n, 128)
    p2 = _prop_call(p1, packed_n, 128)
    out = _final_call(h.reshape(2, N_PAD, 128), p1.reshape(2, N_PAD, 128),
                      p2.reshape(2, N_PAD, 128), Wo, bo.reshape(1, 128),
                      Wm1, bm1.reshape(1, 128), gamma.reshape(1, 128),
                      beta.reshape(1, 128),
                      jnp.pad(Wm2, ((0, 0), (0, 127))),
                      jnp.full((1, 128), bm2[0], dtype=f32))
    return out[:N, :1]
